# baseline (speedup 1.0000x reference)
import functools
import math

import jax
import jax.numpy as jnp
from jax.experimental import pallas as pl
from jax.experimental.pallas import tpu as pltpu
from jax.experimental.pallas import tpu_sc as plsc

_INV_SQRT2 = 1.0 / math.sqrt(2.0)


def _gelu(x):
    return 0.5 * x * (1.0 + jax.lax.erf(x * _INV_SQRT2))


def _bn_coeffs(stats_ref, g_ref, b_ref, cnt):
    mean = stats_ref[:, 0:1] / cnt
    var = stats_ref[:, 1:2] / cnt - mean * mean
    scale = g_ref[...] * jax.lax.rsqrt(var + 1e-5)
    off = b_ref[...] - mean * scale
    return scale, off


def _stats_update(t):
    s = jnp.sum(t, axis=1, keepdims=True)
    ss = jnp.sum(t * t, axis=1, keepdims=True)
    z = jnp.zeros((t.shape[0], 6), jnp.float32)
    return jnp.concatenate([s, ss, z], axis=1)


def _select_body(xyz1_ref, xyz2_ref, idx_ref, w_ref):
    b = pl.program_id(0)

    x1 = xyz1_ref[0]
    x2 = xyz2_ref[0]
    S = x2.shape[1]
    Nb = x1.shape[1]

    n1 = (x1[0:1, :] * x1[0:1, :] + x1[1:2, :] * x1[1:2, :]) \
        + x1[2:3, :] * x1[2:3, :]
    n2 = (x2[0:1, :] * x2[0:1, :] + x2[1:2, :] * x2[1:2, :]) \
        + x2[2:3, :] * x2[2:3, :]
    cross = jax.lax.dot_general(
        x2, x1, (((0,), (0,)), ((), ())),
        preferred_element_type=jnp.float32)
    u = cross - (0.5 * n2).reshape(S, 1)

    iota = jax.lax.broadcasted_iota(jnp.int32, (S, Nb), 0)
    big = jnp.float32(3.0e38)
    work = u
    ams = []
    for j in range(3):
        mx = jnp.max(work, axis=0, keepdims=True)
        sel = work >= mx
        am = jnp.min(jnp.where(sel, iota, S), axis=0, keepdims=True)
        ams.append(am)
        if j < 2:
            work = jnp.where(iota == am, -big, work)

    bx2 = x2.astype(jnp.bfloat16).astype(jnp.float32)
    h0 = n2.astype(jnp.bfloat16).astype(jnp.float32)
    rr = n2 - h0
    h1 = rr.astype(jnp.bfloat16).astype(jnp.float32)
    rr = rr - h1
    h2 = rr.astype(jnp.bfloat16).astype(jnp.float32)
    h3 = (rr - h2).astype(jnp.bfloat16).astype(jnp.float32)
    gmat = jnp.concatenate(
        [bx2, h0, h1, h2, h3, jnp.zeros((1, S), jnp.float32)], axis=0)

    bx1 = x1.astype(jnp.bfloat16).astype(jnp.float32)
    one = jnp.float32(1.0)
    zero = jnp.float32(0.0)
    vals = []
    for j in range(3):
        oh = jnp.where(iota == ams[j], one, zero)
        g = jax.lax.dot_general(
            gmat, oh, (((1,), (0,)), ((), ())),
            preferred_element_type=jnp.float32)
        p0 = bx1[0:1, :] * g[0:1, :]
        p1 = bx1[1:2, :] * g[1:2, :]
        p2 = bx1[2:3, :] * g[2:3, :]
        s1 = p0 + p1
        bv = s1 - p0
        e1 = (p0 - (s1 - bv)) + (p1 - bv)
        s2 = s1 + p2
        bv2 = s2 - s1
        e2 = (s1 - (s2 - bv2)) + (p2 - bv2)
        mm = s2 + (e1 + e2)
        n2sel = ((g[3:4, :] + g[4:5, :]) + g[5:6, :]) + g[6:7, :]
        vals.append(((-2.0 * mm) + n1) + n2sel)

    r0 = 1.0 / (vals[0] + 1e-8)
    r1 = 1.0 / (vals[1] + 1e-8)
    r2 = 1.0 / (vals[2] + 1e-8)
    norm = r0 + r1 + r2
    base = b * S
    izero = jnp.zeros((5, Nb), jnp.int32)
    idx_ref[0] = jnp.concatenate(
        [ams[0] + base, ams[1] + base, ams[2] + base, izero], axis=0)
    fzero = jnp.zeros((5, Nb), jnp.float32)
    w_ref[0] = jnp.concatenate(
        [r0 / norm, r1 / norm, r2 / norm, fzero], axis=0)


def _make_sc_gather(M, Dm, CH):
    info = plsc.get_sparse_core_info()
    NW = info.num_cores * info.num_subcores
    pts_per_w = M // NW
    n_chunks = pts_per_w // CH
    mesh = plsc.VectorSubcoreMesh(core_axis_name="c", subcore_axis_name="s")

    @functools.partial(
        pl.kernel, mesh=mesh,
        out_type=jax.ShapeDtypeStruct((M, Dm), jnp.float32),
        scratch_types=[
            pltpu.VMEM((3 * CH,), jnp.int32),
            pltpu.VMEM((3 * CH, Dm), jnp.float32),
            pltpu.VMEM((3 * CH, 16), jnp.float32),
            pltpu.VMEM((CH, Dm), jnp.float32),
            pltpu.SemaphoreType.DMA,
        ],
    )
    def sc_k(table_hbm, idx_hbm, w16_hbm, out_hbm,
             idx_v, rows_v, w_v, out_v, sem):
        wid = jax.lax.axis_index("s") * info.num_cores + jax.lax.axis_index("c")
        pt_base0 = wid * pts_per_w

        def chunk_body(g, carry):
            pt_base = pt_base0 + g * CH
            ib = pt_base * 3
            pltpu.sync_copy(idx_hbm.at[pl.ds(ib, 3 * CH)], idx_v)
            pltpu.sync_copy(w16_hbm.at[pl.ds(ib, 3 * CH)], w_v)
            pltpu.async_copy(table_hbm.at[idx_v], rows_v, sem).wait()
            for p in range(CH):
                for c in range(Dm // 16):
                    sl = pl.ds(c * 16, 16)
                    acc = (rows_v[3 * p, sl] * w_v[3 * p, :]
                           + rows_v[3 * p + 1, sl] * w_v[3 * p + 1, :])
                    acc = acc + rows_v[3 * p + 2, sl] * w_v[3 * p + 2, :]
                    out_v[p, sl] = acc
            pltpu.sync_copy(out_v, out_hbm.at[pl.ds(pt_base, CH)])
            return carry

        jax.lax.fori_loop(0, n_chunks, chunk_body, 0, unroll=False)

    return sc_k


def _fuse2_body(p1_ref, gi_ref, wfuse_ref, y_ref, stats_ref):
    b = pl.program_id(0)
    nb = pl.program_id(1)

    @pl.when(jnp.logical_and(b == 0, nb == 0))
    def _():
        stats_ref[...] = jnp.zeros_like(stats_ref)

    D = p1_ref.shape[1]
    wf = wfuse_ref[...]
    y = jax.lax.dot_general(
        wf[:, :D], p1_ref[0], (((1,), (0,)), ((), ())),
        preferred_element_type=jnp.float32)
    y = y + jax.lax.dot_general(
        wf[:, D:], gi_ref[...], (((1,), (1,)), ((), ())),
        preferred_element_type=jnp.float32)
    y_ref[0] = y
    stats_ref[...] += _stats_update(y)


def _mlp_body(cnt, xin_ref, stats_in_ref, g_ref, b_ref, w_ref,
              out_ref, stats_out_ref):
    b = pl.program_id(0)
    nb = pl.program_id(1)

    @pl.when(jnp.logical_and(b == 0, nb == 0))
    def _():
        stats_out_ref[...] = jnp.zeros_like(stats_out_ref)

    scale, off = _bn_coeffs(stats_in_ref, g_ref, b_ref, cnt)
    x = _gelu(xin_ref[0] * scale + off)
    t = jax.lax.dot_general(
        w_ref[...], x, (((1,), (0,)), ((), ())),
        preferred_element_type=jnp.float32)
    out_ref[0] = t
    stats_out_ref[...] += _stats_update(t)


def _final_body(cnt, y_ref, stats_y_ref, gf_ref, bf_ref,
                t2_ref, stats2_ref, g2_ref, b2_ref, out_ref):
    scale_f, off_f = _bn_coeffs(stats_y_ref, gf_ref, bf_ref, cnt)
    x = _gelu(y_ref[0] * scale_f + off_f)
    scale2, off2 = _bn_coeffs(stats2_ref, g2_ref, b2_ref, cnt)
    h = t2_ref[0] * scale2 + off2
    out_ref[0] = _gelu(h + x)


def kernel(xyz1, xyz2, points1, points2, W_fuse, g_fuse, b_fuse,
           W1, g1, b1, W2, g2, b2):
    B, N, _ = xyz1.shape
    S = xyz2.shape[1]
    D = points1.shape[1]
    C = W_fuse.shape[0]
    cnt = float(B * N)

    Nb = 512 if N % 512 == 0 else N
    NB = N // Nb

    xyz1t = jnp.transpose(xyz1, (0, 2, 1))
    xyz2t = jnp.transpose(xyz2, (0, 2, 1))
    col = lambda v: v.reshape(C, 1)

    f32 = jnp.float32
    grid = (B, NB)

    blk_n = pl.BlockSpec((1, C, Nb), lambda b, nb: (b, 0, nb))
    blk_full = lambda shape: pl.BlockSpec(shape, lambda b, nb: (0,) * len(shape))
    blk_stats = pl.BlockSpec((C, 8), lambda b, nb: (0, 0))

    idx_out, w_out = pl.pallas_call(
        _select_body,
        grid=grid,
        in_specs=[
            pl.BlockSpec((1, 3, Nb), lambda b, nb: (b, 0, nb)),
            pl.BlockSpec((1, 3, S), lambda b, nb: (b, 0, 0)),
        ],
        out_specs=[pl.BlockSpec((1, 8, Nb), lambda b, nb: (b, 0, nb)),
                   pl.BlockSpec((1, 8, Nb), lambda b, nb: (b, 0, nb))],
        out_shape=[
            jax.ShapeDtypeStruct((B, 8, N), jnp.int32),
            jax.ShapeDtypeStruct((B, 8, N), f32),
        ],
    )(xyz1t, xyz2t)

    M = B * N
    idx_flat = idx_out[:, :3, :].transpose(0, 2, 1).reshape(3 * M)
    w3 = w_out[:, :3, :].transpose(0, 2, 1).reshape(3 * M, 1)
    w16 = jnp.broadcast_to(w3, (3 * M, 16))
    table = jnp.transpose(points2, (0, 2, 1)).reshape(B * S, D)

    interp = _make_sc_gather(M, D, 16)(table, idx_flat, w16)

    y, stats_y = pl.pallas_call(
        _fuse2_body,
        grid=grid,
        in_specs=[
            blk_n,
            pl.BlockSpec((Nb, D), lambda b, nb: (b * NB + nb, 0)),
            blk_full((C, 2 * D)),
        ],
        out_specs=[blk_n, blk_stats],
        out_shape=[
            jax.ShapeDtypeStruct((B, C, N), f32),
            jax.ShapeDtypeStruct((C, 8), f32),
        ],
    )(points1, interp, W_fuse)

    Nb2 = 2048 if N % 2048 == 0 else Nb
    grid2 = (B, N // Nb2)
    blk_n2 = pl.BlockSpec((1, C, Nb2), lambda b, nb: (b, 0, nb))

    mlp = functools.partial(_mlp_body, cnt)
    mlp_call = lambda xin, stats, g, b, w: pl.pallas_call(
        mlp,
        grid=grid2,
        in_specs=[
            blk_n2,
            blk_stats,
            blk_full((C, 1)),
            blk_full((C, 1)),
            blk_full((C, C)),
        ],
        out_specs=[blk_n2, blk_stats],
        out_shape=[
            jax.ShapeDtypeStruct((B, C, N), f32),
            jax.ShapeDtypeStruct((C, 8), f32),
        ],
    )(xin, stats, col(g), col(b), w)

    t1, stats_1 = mlp_call(y, stats_y, g_fuse, b_fuse, W1)
    t2, stats_2 = mlp_call(t1, stats_1, g1, b1, W2)

    out = pl.pallas_call(
        functools.partial(_final_body, cnt),
        grid=grid2,
        in_specs=[
            blk_n2,
            blk_stats,
            blk_full((C, 1)),
            blk_full((C, 1)),
            blk_n2,
            blk_stats,
            blk_full((C, 1)),
            blk_full((C, 1)),
        ],
        out_specs=blk_n2,
        out_shape=jax.ShapeDtypeStruct((B, C, N), f32),
    )(y, stats_y, col(g_fuse), col(b_fuse), t2, stats_2, col(g2), col(b2))

    return out


# speedup vs baseline: 1.0896x; 1.0896x over previous
import functools
import math

import jax
import jax.numpy as jnp
from jax.experimental import pallas as pl
from jax.experimental.pallas import tpu as pltpu
from jax.experimental.pallas import tpu_sc as plsc

_INV_SQRT2 = 1.0 / math.sqrt(2.0)


def _gelu(x):
    return 0.5 * x * (1.0 + jax.lax.erf(x * _INV_SQRT2))


def _bn_coeffs(stats_ref, g_ref, b_ref, cnt):
    mean = stats_ref[:, 0:1] / cnt
    var = stats_ref[:, 1:2] / cnt - mean * mean
    scale = g_ref[...] * jax.lax.rsqrt(var + 1e-5)
    off = b_ref[...] - mean * scale
    return scale, off


def _stats_update(t):
    s = jnp.sum(t, axis=1, keepdims=True)
    ss = jnp.sum(t * t, axis=1, keepdims=True)
    z = jnp.zeros((t.shape[0], 6), jnp.float32)
    return jnp.concatenate([s, ss, z], axis=1)


def _select_body(xyz1_ref, xyz2_ref, idx_ref, w_ref):
    b = pl.program_id(0)

    x1 = xyz1_ref[0]
    x2 = xyz2_ref[0]
    S = x2.shape[1]
    Nb = x1.shape[1]

    n1 = (x1[0:1, :] * x1[0:1, :] + x1[1:2, :] * x1[1:2, :]) \
        + x1[2:3, :] * x1[2:3, :]
    n2 = (x2[0:1, :] * x2[0:1, :] + x2[1:2, :] * x2[1:2, :]) \
        + x2[2:3, :] * x2[2:3, :]
    cross = jax.lax.dot_general(
        x2, x1, (((0,), (0,)), ((), ())),
        preferred_element_type=jnp.float32)
    u = cross - (0.5 * n2).reshape(S, 1)

    iota = jax.lax.broadcasted_iota(jnp.int32, (S, Nb), 0)
    big = jnp.float32(3.0e38)
    work = u
    ams = []
    for j in range(3):
        mx = jnp.max(work, axis=0, keepdims=True)
        sel = work >= mx
        am = jnp.min(jnp.where(sel, iota, S), axis=0, keepdims=True)
        ams.append(am)
        if j < 2:
            work = jnp.where(iota == am, -big, work)

    bx2 = x2.astype(jnp.bfloat16).astype(jnp.float32)
    h0 = n2.astype(jnp.bfloat16).astype(jnp.float32)
    rr = n2 - h0
    h1 = rr.astype(jnp.bfloat16).astype(jnp.float32)
    rr = rr - h1
    h2 = rr.astype(jnp.bfloat16).astype(jnp.float32)
    h3 = (rr - h2).astype(jnp.bfloat16).astype(jnp.float32)
    gmat = jnp.concatenate(
        [bx2, h0, h1, h2, h3, jnp.zeros((1, S), jnp.float32)], axis=0)

    bx1 = x1.astype(jnp.bfloat16).astype(jnp.float32)
    one = jnp.float32(1.0)
    zero = jnp.float32(0.0)
    vals = []
    for j in range(3):
        oh = jnp.where(iota == ams[j], one, zero)
        g = jax.lax.dot_general(
            gmat, oh, (((1,), (0,)), ((), ())),
            preferred_element_type=jnp.float32)
        p0 = bx1[0:1, :] * g[0:1, :]
        p1 = bx1[1:2, :] * g[1:2, :]
        p2 = bx1[2:3, :] * g[2:3, :]
        s1 = p0 + p1
        bv = s1 - p0
        e1 = (p0 - (s1 - bv)) + (p1 - bv)
        s2 = s1 + p2
        bv2 = s2 - s1
        e2 = (s1 - (s2 - bv2)) + (p2 - bv2)
        mm = s2 + (e1 + e2)
        n2sel = ((g[3:4, :] + g[4:5, :]) + g[5:6, :]) + g[6:7, :]
        vals.append(((-2.0 * mm) + n1) + n2sel)

    r0 = 1.0 / (vals[0] + 1e-8)
    r1 = 1.0 / (vals[1] + 1e-8)
    r2 = 1.0 / (vals[2] + 1e-8)
    norm = r0 + r1 + r2
    base = b * S
    izero = jnp.zeros((5, Nb), jnp.int32)
    idx_ref[0] = jnp.concatenate(
        [ams[0] + base, ams[1] + base, ams[2] + base, izero], axis=0)
    fzero = jnp.zeros((5, Nb), jnp.float32)
    w_ref[0] = jnp.concatenate(
        [r0 / norm, r1 / norm, r2 / norm, fzero], axis=0)


def _make_sc_gather(M, Dm, CH):
    info = plsc.get_sparse_core_info()
    NW = info.num_cores * info.num_subcores
    pts_per_w = M // NW
    n_chunks = pts_per_w // CH
    mesh = plsc.VectorSubcoreMesh(core_axis_name="c", subcore_axis_name="s")

    @functools.partial(
        pl.kernel, mesh=mesh,
        out_type=jax.ShapeDtypeStruct((M, Dm), jnp.float32),
        scratch_types=[
            pltpu.VMEM((3 * CH,), jnp.int32),
            pltpu.VMEM((3 * CH,), jnp.int32),
            pltpu.VMEM((3 * CH, Dm), jnp.float32),
            pltpu.VMEM((3 * CH, Dm), jnp.float32),
            pltpu.VMEM((3 * CH, 16), jnp.float32),
            pltpu.VMEM((3 * CH, 16), jnp.float32),
            pltpu.VMEM((CH, Dm), jnp.float32),
            pltpu.SemaphoreType.DMA,
            pltpu.SemaphoreType.DMA,
            pltpu.SemaphoreType.DMA,
            pltpu.SemaphoreType.DMA,
        ],
    )
    def sc_k(table_hbm, idx_hbm, w16_hbm, out_hbm,
             idx_v0, idx_v1, rows_v0, rows_v1, w_v0, w_v1, out_v,
             semr0, semr1, semw0, semw1):
        wid = jax.lax.axis_index("s") * info.num_cores + jax.lax.axis_index("c")
        pt_base0 = wid * pts_per_w
        idx_v = (idx_v0, idx_v1)
        rows_v = (rows_v0, rows_v1)
        w_v = (w_v0, w_v1)
        semr = (semr0, semr1)
        semw = (semw0, semw1)

        def fire(c, buf):
            ib = (pt_base0 + c * CH) * 3
            pltpu.sync_copy(idx_hbm.at[pl.ds(ib, 3 * CH)], idx_v[buf])
            pltpu.async_copy(table_hbm.at[idx_v[buf]], rows_v[buf], semr[buf])
            pltpu.async_copy(w16_hbm.at[pl.ds(ib, 3 * CH)], w_v[buf], semw[buf])

        fire(0, 0)

        def pair_body(gp, carry):
            for b in (0, 1):
                cur = 2 * gp + b
                pltpu.make_async_copy(
                    table_hbm.at[idx_v[b]], rows_v[b], semr[b]).wait()
                pltpu.make_async_copy(
                    w16_hbm.at[pl.ds(0, 3 * CH)], w_v[b], semw[b]).wait()

                nxt = cur + 1

                @pl.when(nxt < n_chunks)
                def _():
                    fire(nxt, 1 - b)

                rv = rows_v[b]
                wv = w_v[b]
                for p in range(CH):
                    for c in range(Dm // 16):
                        sl = pl.ds(c * 16, 16)
                        acc = (rv[3 * p, sl] * wv[3 * p, :]
                               + rv[3 * p + 1, sl] * wv[3 * p + 1, :])
                        acc = acc + rv[3 * p + 2, sl] * wv[3 * p + 2, :]
                        out_v[p, sl] = acc
                pltpu.sync_copy(
                    out_v, out_hbm.at[pl.ds(pt_base0 + cur * CH, CH)])
            return carry

        jax.lax.fori_loop(0, n_chunks // 2, pair_body, 0, unroll=False)

    return sc_k


def _fuse2_body(p1_ref, gi_ref, wfuse_ref, y_ref, stats_ref):
    b = pl.program_id(0)
    nb = pl.program_id(1)

    @pl.when(jnp.logical_and(b == 0, nb == 0))
    def _():
        stats_ref[...] = jnp.zeros_like(stats_ref)

    D = p1_ref.shape[1]
    wf = wfuse_ref[...]
    y = jax.lax.dot_general(
        wf[:, :D], p1_ref[0], (((1,), (0,)), ((), ())),
        preferred_element_type=jnp.float32)
    y = y + jax.lax.dot_general(
        wf[:, D:], gi_ref[...], (((1,), (1,)), ((), ())),
        preferred_element_type=jnp.float32)
    y_ref[0] = y
    stats_ref[...] += _stats_update(y)


def _mlp_body(cnt, xin_ref, stats_in_ref, g_ref, b_ref, w_ref,
              out_ref, stats_out_ref):
    b = pl.program_id(0)
    nb = pl.program_id(1)

    @pl.when(jnp.logical_and(b == 0, nb == 0))
    def _():
        stats_out_ref[...] = jnp.zeros_like(stats_out_ref)

    scale, off = _bn_coeffs(stats_in_ref, g_ref, b_ref, cnt)
    x = _gelu(xin_ref[0] * scale + off)
    t = jax.lax.dot_general(
        w_ref[...], x, (((1,), (0,)), ((), ())),
        preferred_element_type=jnp.float32)
    out_ref[0] = t
    stats_out_ref[...] += _stats_update(t)


def _final_body(cnt, y_ref, stats_y_ref, gf_ref, bf_ref,
                t2_ref, stats2_ref, g2_ref, b2_ref, out_ref):
    scale_f, off_f = _bn_coeffs(stats_y_ref, gf_ref, bf_ref, cnt)
    x = _gelu(y_ref[0] * scale_f + off_f)
    scale2, off2 = _bn_coeffs(stats2_ref, g2_ref, b2_ref, cnt)
    h = t2_ref[0] * scale2 + off2
    out_ref[0] = _gelu(h + x)


def kernel(xyz1, xyz2, points1, points2, W_fuse, g_fuse, b_fuse,
           W1, g1, b1, W2, g2, b2):
    B, N, _ = xyz1.shape
    S = xyz2.shape[1]
    D = points1.shape[1]
    C = W_fuse.shape[0]
    cnt = float(B * N)

    Nb = 512 if N % 512 == 0 else N
    NB = N // Nb

    xyz1t = jnp.transpose(xyz1, (0, 2, 1))
    xyz2t = jnp.transpose(xyz2, (0, 2, 1))
    col = lambda v: v.reshape(C, 1)

    f32 = jnp.float32
    grid = (B, NB)

    blk_n = pl.BlockSpec((1, C, Nb), lambda b, nb: (b, 0, nb))
    blk_full = lambda shape: pl.BlockSpec(shape, lambda b, nb: (0,) * len(shape))
    blk_stats = pl.BlockSpec((C, 8), lambda b, nb: (0, 0))

    idx_out, w_out = pl.pallas_call(
        _select_body,
        grid=grid,
        in_specs=[
            pl.BlockSpec((1, 3, Nb), lambda b, nb: (b, 0, nb)),
            pl.BlockSpec((1, 3, S), lambda b, nb: (b, 0, 0)),
        ],
        out_specs=[pl.BlockSpec((1, 8, Nb), lambda b, nb: (b, 0, nb)),
                   pl.BlockSpec((1, 8, Nb), lambda b, nb: (b, 0, nb))],
        out_shape=[
            jax.ShapeDtypeStruct((B, 8, N), jnp.int32),
            jax.ShapeDtypeStruct((B, 8, N), f32),
        ],
    )(xyz1t, xyz2t)

    M = B * N
    idx_flat = idx_out[:, :3, :].transpose(0, 2, 1).reshape(3 * M)
    w3 = w_out[:, :3, :].transpose(0, 2, 1).reshape(3 * M, 1)
    w16 = jnp.broadcast_to(w3, (3 * M, 16))
    table = jnp.transpose(points2, (0, 2, 1)).reshape(B * S, D)

    interp = _make_sc_gather(M, D, 16)(table, idx_flat, w16)

    y, stats_y = pl.pallas_call(
        _fuse2_body,
        grid=grid,
        in_specs=[
            blk_n,
            pl.BlockSpec((Nb, D), lambda b, nb: (b * NB + nb, 0)),
            blk_full((C, 2 * D)),
        ],
        out_specs=[blk_n, blk_stats],
        out_shape=[
            jax.ShapeDtypeStruct((B, C, N), f32),
            jax.ShapeDtypeStruct((C, 8), f32),
        ],
    )(points1, interp, W_fuse)

    Nb2 = 2048 if N % 2048 == 0 else Nb
    grid2 = (B, N // Nb2)
    blk_n2 = pl.BlockSpec((1, C, Nb2), lambda b, nb: (b, 0, nb))

    mlp = functools.partial(_mlp_body, cnt)
    mlp_call = lambda xin, stats, g, b, w: pl.pallas_call(
        mlp,
        grid=grid2,
        in_specs=[
            blk_n2,
            blk_stats,
            blk_full((C, 1)),
            blk_full((C, 1)),
            blk_full((C, C)),
        ],
        out_specs=[blk_n2, blk_stats],
        out_shape=[
            jax.ShapeDtypeStruct((B, C, N), f32),
            jax.ShapeDtypeStruct((C, 8), f32),
        ],
    )(xin, stats, col(g), col(b), w)

    t1, stats_1 = mlp_call(y, stats_y, g_fuse, b_fuse, W1)
    t2, stats_2 = mlp_call(t1, stats_1, g1, b1, W2)

    out = pl.pallas_call(
        functools.partial(_final_body, cnt),
        grid=grid2,
        in_specs=[
            blk_n2,
            blk_stats,
            blk_full((C, 1)),
            blk_full((C, 1)),
            blk_n2,
            blk_stats,
            blk_full((C, 1)),
            blk_full((C, 1)),
        ],
        out_specs=blk_n2,
        out_shape=jax.ShapeDtypeStruct((B, C, N), f32),
    )(y, stats_y, col(g_fuse), col(b_fuse), t2, stats_2, col(g2), col(b2))

    return out


# speedup vs baseline: 1.3817x; 1.2681x over previous
import functools
import math

import jax
import jax.numpy as jnp
from jax.experimental import pallas as pl
from jax.experimental.pallas import tpu as pltpu
from jax.experimental.pallas import tpu_sc as plsc

_INV_SQRT2 = 1.0 / math.sqrt(2.0)


def _gelu(x):
    return 0.5 * x * (1.0 + jax.lax.erf(x * _INV_SQRT2))


def _bn_coeffs(stats_ref, g_ref, b_ref, cnt):
    mean = stats_ref[:, 0:1] / cnt
    var = stats_ref[:, 1:2] / cnt - mean * mean
    scale = g_ref[...] * jax.lax.rsqrt(var + 1e-5)
    off = b_ref[...] - mean * scale
    return scale, off


def _stats_update(t):
    s = jnp.sum(t, axis=1, keepdims=True)
    ss = jnp.sum(t * t, axis=1, keepdims=True)
    z = jnp.zeros((t.shape[0], 6), jnp.float32)
    return jnp.concatenate([s, ss, z], axis=1)


def _select_body(xyz1_ref, xyz2_ref, idx_ref, w_ref):
    b = pl.program_id(0)

    x1 = xyz1_ref[0]
    x2 = xyz2_ref[0]
    S = x2.shape[1]
    Nb = x1.shape[1]

    n1 = (x1[0:1, :] * x1[0:1, :] + x1[1:2, :] * x1[1:2, :]) \
        + x1[2:3, :] * x1[2:3, :]
    n2 = (x2[0:1, :] * x2[0:1, :] + x2[1:2, :] * x2[1:2, :]) \
        + x2[2:3, :] * x2[2:3, :]
    cross = jax.lax.dot_general(
        x2, x1, (((0,), (0,)), ((), ())),
        preferred_element_type=jnp.float32)
    u = cross - (0.5 * n2).reshape(S, 1)

    iota = jax.lax.broadcasted_iota(jnp.int32, (S, Nb), 0)
    big = jnp.float32(3.0e38)
    work = u
    ams = []
    for j in range(3):
        mx = jnp.max(work, axis=0, keepdims=True)
        sel = work >= mx
        am = jnp.min(jnp.where(sel, iota, S), axis=0, keepdims=True)
        ams.append(am)
        if j < 2:
            work = jnp.where(iota == am, -big, work)

    bx2 = x2.astype(jnp.bfloat16).astype(jnp.float32)
    h0 = n2.astype(jnp.bfloat16).astype(jnp.float32)
    rr = n2 - h0
    h1 = rr.astype(jnp.bfloat16).astype(jnp.float32)
    rr = rr - h1
    h2 = rr.astype(jnp.bfloat16).astype(jnp.float32)
    h3 = (rr - h2).astype(jnp.bfloat16).astype(jnp.float32)
    gmat = jnp.concatenate(
        [bx2, h0, h1, h2, h3, jnp.zeros((1, S), jnp.float32)], axis=0)

    bx1 = x1.astype(jnp.bfloat16).astype(jnp.float32)
    one = jnp.float32(1.0)
    zero = jnp.float32(0.0)
    vals = []
    for j in range(3):
        oh = jnp.where(iota == ams[j], one, zero)
        g = jax.lax.dot_general(
            gmat, oh, (((1,), (0,)), ((), ())),
            preferred_element_type=jnp.float32)
        p0 = bx1[0:1, :] * g[0:1, :]
        p1 = bx1[1:2, :] * g[1:2, :]
        p2 = bx1[2:3, :] * g[2:3, :]
        s1 = p0 + p1
        bv = s1 - p0
        e1 = (p0 - (s1 - bv)) + (p1 - bv)
        s2 = s1 + p2
        bv2 = s2 - s1
        e2 = (s1 - (s2 - bv2)) + (p2 - bv2)
        mm = s2 + (e1 + e2)
        n2sel = ((g[3:4, :] + g[4:5, :]) + g[5:6, :]) + g[6:7, :]
        vals.append(((-2.0 * mm) + n1) + n2sel)

    r0 = 1.0 / (vals[0] + 1e-8)
    r1 = 1.0 / (vals[1] + 1e-8)
    r2 = 1.0 / (vals[2] + 1e-8)
    norm = r0 + r1 + r2
    base = b * S
    izero = jnp.zeros((5, Nb), jnp.int32)
    idx_ref[0] = jnp.concatenate(
        [ams[0] + base, ams[1] + base, ams[2] + base, izero], axis=0)
    fzero = jnp.zeros((5, Nb), jnp.float32)
    w_ref[0] = jnp.concatenate(
        [r0 / norm, r1 / norm, r2 / norm, fzero], axis=0)


def _make_sc_gather(M, Dm, CH):
    info = plsc.get_sparse_core_info()
    NW = info.num_cores * info.num_subcores
    pts_per_w = M // NW
    n_chunks = pts_per_w // CH
    mesh = plsc.VectorSubcoreMesh(core_axis_name="c", subcore_axis_name="s")

    @functools.partial(
        pl.kernel, mesh=mesh,
        out_type=jax.ShapeDtypeStruct((M, Dm), jnp.float32),
        scratch_types=[
            pltpu.VMEM((3 * CH,), jnp.int32),
            pltpu.VMEM((3 * CH,), jnp.int32),
            pltpu.VMEM((3 * CH, Dm), jnp.float32),
            pltpu.VMEM((3 * CH, Dm), jnp.float32),
            pltpu.VMEM((3 * CH, 16), jnp.float32),
            pltpu.VMEM((3 * CH, 16), jnp.float32),
            pltpu.VMEM((CH, Dm), jnp.float32),
            pltpu.SemaphoreType.DMA,
            pltpu.SemaphoreType.DMA,
            pltpu.SemaphoreType.DMA,
            pltpu.SemaphoreType.DMA,
        ],
    )
    def sc_k(table_hbm, idx_hbm, w16_hbm, out_hbm,
             idx_v0, idx_v1, rows_v0, rows_v1, w_v0, w_v1, out_v,
             semr0, semr1, semw0, semw1):
        wid = jax.lax.axis_index("s") * info.num_cores + jax.lax.axis_index("c")
        pt_base0 = wid * pts_per_w
        idx_v = (idx_v0, idx_v1)
        rows_v = (rows_v0, rows_v1)
        w_v = (w_v0, w_v1)
        semr = (semr0, semr1)
        semw = (semw0, semw1)

        def fire(c, buf):
            ib = (pt_base0 + c * CH) * 3
            pltpu.sync_copy(idx_hbm.at[pl.ds(ib, 3 * CH)], idx_v[buf])
            pltpu.async_copy(table_hbm.at[idx_v[buf]], rows_v[buf], semr[buf])
            pltpu.async_copy(w16_hbm.at[pl.ds(ib, 3 * CH)], w_v[buf], semw[buf])

        fire(0, 0)

        def pair_body(gp, carry):
            for b in (0, 1):
                cur = 2 * gp + b
                pltpu.make_async_copy(
                    table_hbm.at[idx_v[b]], rows_v[b], semr[b]).wait()
                pltpu.make_async_copy(
                    w16_hbm.at[pl.ds(0, 3 * CH)], w_v[b], semw[b]).wait()

                nxt = cur + 1

                @pl.when(nxt < n_chunks)
                def _():
                    fire(nxt, 1 - b)

                rv = rows_v[b]
                wv = w_v[b]
                for p in range(CH):
                    w0 = wv[3 * p, :]
                    w1 = wv[3 * p + 1, :]
                    w2 = wv[3 * p + 2, :]
                    for c in range(Dm // 16):
                        sl = pl.ds(c * 16, 16)
                        acc = (rv[3 * p, sl] * w0
                               + rv[3 * p + 1, sl] * w1)
                        acc = acc + rv[3 * p + 2, sl] * w2
                        out_v[p, sl] = acc
                pltpu.sync_copy(
                    out_v, out_hbm.at[pl.ds(pt_base0 + cur * CH, CH)])
            return carry

        jax.lax.fori_loop(0, n_chunks // 2, pair_body, 0, unroll=False)

    return sc_k


def _fuse2_body(p1_ref, gi_ref, wfuse_ref, y_ref, stats_ref):
    b = pl.program_id(0)
    nb = pl.program_id(1)

    @pl.when(jnp.logical_and(b == 0, nb == 0))
    def _():
        stats_ref[...] = jnp.zeros_like(stats_ref)

    D = p1_ref.shape[1]
    wf = wfuse_ref[...]
    y = jax.lax.dot_general(
        wf[:, :D], p1_ref[0], (((1,), (0,)), ((), ())),
        preferred_element_type=jnp.float32)
    y = y + jax.lax.dot_general(
        wf[:, D:], gi_ref[...], (((1,), (1,)), ((), ())),
        preferred_element_type=jnp.float32)
    y_ref[0] = y
    stats_ref[...] += _stats_update(y)


def _mlp_body(cnt, xin_ref, stats_in_ref, g_ref, b_ref, w_ref,
              out_ref, stats_out_ref):
    b = pl.program_id(0)
    nb = pl.program_id(1)

    @pl.when(jnp.logical_and(b == 0, nb == 0))
    def _():
        stats_out_ref[...] = jnp.zeros_like(stats_out_ref)

    scale, off = _bn_coeffs(stats_in_ref, g_ref, b_ref, cnt)
    x = _gelu(xin_ref[0] * scale + off)
    t = jax.lax.dot_general(
        w_ref[...], x, (((1,), (0,)), ((), ())),
        preferred_element_type=jnp.float32)
    out_ref[0] = t
    stats_out_ref[...] += _stats_update(t)


def _final_body(cnt, y_ref, stats_y_ref, gf_ref, bf_ref,
                t2_ref, stats2_ref, g2_ref, b2_ref, out_ref):
    scale_f, off_f = _bn_coeffs(stats_y_ref, gf_ref, bf_ref, cnt)
    x = _gelu(y_ref[0] * scale_f + off_f)
    scale2, off2 = _bn_coeffs(stats2_ref, g2_ref, b2_ref, cnt)
    h = t2_ref[0] * scale2 + off2
    out_ref[0] = _gelu(h + x)


def kernel(xyz1, xyz2, points1, points2, W_fuse, g_fuse, b_fuse,
           W1, g1, b1, W2, g2, b2):
    B, N, _ = xyz1.shape
    S = xyz2.shape[1]
    D = points1.shape[1]
    C = W_fuse.shape[0]
    cnt = float(B * N)

    Nb = 512 if N % 512 == 0 else N
    NB = N // Nb

    xyz1t = jnp.transpose(xyz1, (0, 2, 1))
    xyz2t = jnp.transpose(xyz2, (0, 2, 1))
    col = lambda v: v.reshape(C, 1)

    f32 = jnp.float32
    grid = (B, NB)

    blk_n = pl.BlockSpec((1, C, Nb), lambda b, nb: (b, 0, nb))
    blk_full = lambda shape: pl.BlockSpec(shape, lambda b, nb: (0,) * len(shape))
    blk_stats = pl.BlockSpec((C, 8), lambda b, nb: (0, 0))

    idx_out, w_out = pl.pallas_call(
        _select_body,
        grid=grid,
        in_specs=[
            pl.BlockSpec((1, 3, Nb), lambda b, nb: (b, 0, nb)),
            pl.BlockSpec((1, 3, S), lambda b, nb: (b, 0, 0)),
        ],
        out_specs=[pl.BlockSpec((1, 8, Nb), lambda b, nb: (b, 0, nb)),
                   pl.BlockSpec((1, 8, Nb), lambda b, nb: (b, 0, nb))],
        out_shape=[
            jax.ShapeDtypeStruct((B, 8, N), jnp.int32),
            jax.ShapeDtypeStruct((B, 8, N), f32),
        ],
    )(xyz1t, xyz2t)

    M = B * N
    idx_flat = idx_out[:, :3, :].transpose(0, 2, 1).reshape(3 * M)
    w3 = w_out[:, :3, :].transpose(0, 2, 1).reshape(3 * M, 1)
    w16 = jnp.broadcast_to(w3, (3 * M, 16))
    table = jnp.transpose(points2, (0, 2, 1)).reshape(B * S, D)

    interp = _make_sc_gather(M, D, 16)(table, idx_flat, w16)

    y, stats_y = pl.pallas_call(
        _fuse2_body,
        grid=grid,
        in_specs=[
            blk_n,
            pl.BlockSpec((Nb, D), lambda b, nb: (b * NB + nb, 0)),
            blk_full((C, 2 * D)),
        ],
        out_specs=[blk_n, blk_stats],
        out_shape=[
            jax.ShapeDtypeStruct((B, C, N), f32),
            jax.ShapeDtypeStruct((C, 8), f32),
        ],
    )(points1, interp, W_fuse)

    Nb2 = 2048 if N % 2048 == 0 else Nb
    grid2 = (B, N // Nb2)
    blk_n2 = pl.BlockSpec((1, C, Nb2), lambda b, nb: (b, 0, nb))

    mlp = functools.partial(_mlp_body, cnt)
    mlp_call = lambda xin, stats, g, b, w: pl.pallas_call(
        mlp,
        grid=grid2,
        in_specs=[
            blk_n2,
            blk_stats,
            blk_full((C, 1)),
            blk_full((C, 1)),
            blk_full((C, C)),
        ],
        out_specs=[blk_n2, blk_stats],
        out_shape=[
            jax.ShapeDtypeStruct((B, C, N), f32),
            jax.ShapeDtypeStruct((C, 8), f32),
        ],
    )(xin, stats, col(g), col(b), w)

    t1, stats_1 = mlp_call(y, stats_y, g_fuse, b_fuse, W1)
    t2, stats_2 = mlp_call(t1, stats_1, g1, b1, W2)

    out = pl.pallas_call(
        functools.partial(_final_body, cnt),
        grid=grid2,
        in_specs=[
            blk_n2,
            blk_stats,
            blk_full((C, 1)),
            blk_full((C, 1)),
            blk_n2,
            blk_stats,
            blk_full((C, 1)),
            blk_full((C, 1)),
        ],
        out_specs=blk_n2,
        out_shape=jax.ShapeDtypeStruct((B, C, N), f32),
    )(y, stats_y, col(g_fuse), col(b_fuse), t2, stats_2, col(g2), col(b2))

    return out


# speedup vs baseline: 1.4641x; 1.0596x over previous
import functools
import math

import jax
import jax.numpy as jnp
from jax.experimental import pallas as pl
from jax.experimental.pallas import tpu as pltpu
from jax.experimental.pallas import tpu_sc as plsc

_INV_SQRT2 = 1.0 / math.sqrt(2.0)


def _gelu(x):
    return 0.5 * x * (1.0 + jax.lax.erf(x * _INV_SQRT2))


def _bn_coeffs(stats_ref, g_ref, b_ref, cnt):
    mean = stats_ref[:, 0:1] / cnt
    var = stats_ref[:, 1:2] / cnt - mean * mean
    scale = g_ref[...] * jax.lax.rsqrt(var + 1e-5)
    off = b_ref[...] - mean * scale
    return scale, off


def _stats_update(t):
    s = jnp.sum(t, axis=1, keepdims=True)
    ss = jnp.sum(t * t, axis=1, keepdims=True)
    z = jnp.zeros((t.shape[0], 6), jnp.float32)
    return jnp.concatenate([s, ss, z], axis=1)


def _select_body(xyz1_ref, xyz2_ref, idx_ref, w_ref):
    b = pl.program_id(0)

    x1 = xyz1_ref[0]
    x2 = xyz2_ref[0]
    S = x2.shape[1]
    Nb = x1.shape[1]

    n1 = (x1[0:1, :] * x1[0:1, :] + x1[1:2, :] * x1[1:2, :]) \
        + x1[2:3, :] * x1[2:3, :]
    n2 = (x2[0:1, :] * x2[0:1, :] + x2[1:2, :] * x2[1:2, :]) \
        + x2[2:3, :] * x2[2:3, :]
    cross = jax.lax.dot_general(
        x2, x1, (((0,), (0,)), ((), ())),
        preferred_element_type=jnp.float32)
    u = cross - (0.5 * n2).reshape(S, 1)

    iota = jax.lax.broadcasted_iota(jnp.int32, (S, Nb), 0)
    big = jnp.float32(3.0e38)
    work = u
    ams = []
    for j in range(3):
        mx = jnp.max(work, axis=0, keepdims=True)
        sel = work >= mx
        am = jnp.min(jnp.where(sel, iota, S), axis=0, keepdims=True)
        ams.append(am)
        if j < 2:
            work = jnp.where(iota == am, -big, work)

    bx2 = x2.astype(jnp.bfloat16).astype(jnp.float32)
    h0 = n2.astype(jnp.bfloat16).astype(jnp.float32)
    rr = n2 - h0
    h1 = rr.astype(jnp.bfloat16).astype(jnp.float32)
    rr = rr - h1
    h2 = rr.astype(jnp.bfloat16).astype(jnp.float32)
    h3 = (rr - h2).astype(jnp.bfloat16).astype(jnp.float32)
    gmat = jnp.concatenate(
        [bx2, h0, h1, h2, h3, jnp.zeros((1, S), jnp.float32)], axis=0)

    bx1 = x1.astype(jnp.bfloat16).astype(jnp.float32)
    one = jnp.float32(1.0)
    zero = jnp.float32(0.0)
    vals = []
    for j in range(3):
        oh = jnp.where(iota == ams[j], one, zero)
        g = jax.lax.dot_general(
            gmat, oh, (((1,), (0,)), ((), ())),
            preferred_element_type=jnp.float32)
        p0 = bx1[0:1, :] * g[0:1, :]
        p1 = bx1[1:2, :] * g[1:2, :]
        p2 = bx1[2:3, :] * g[2:3, :]
        s1 = p0 + p1
        bv = s1 - p0
        e1 = (p0 - (s1 - bv)) + (p1 - bv)
        s2 = s1 + p2
        bv2 = s2 - s1
        e2 = (s1 - (s2 - bv2)) + (p2 - bv2)
        mm = s2 + (e1 + e2)
        n2sel = ((g[3:4, :] + g[4:5, :]) + g[5:6, :]) + g[6:7, :]
        vals.append(((-2.0 * mm) + n1) + n2sel)

    r0 = 1.0 / (vals[0] + 1e-8)
    r1 = 1.0 / (vals[1] + 1e-8)
    r2 = 1.0 / (vals[2] + 1e-8)
    norm = r0 + r1 + r2
    base = b * S
    izero = jnp.zeros((5, Nb), jnp.int32)
    idx_ref[0] = jnp.concatenate(
        [ams[0] + base, ams[1] + base, ams[2] + base, izero], axis=0)
    fzero = jnp.zeros((5, Nb), jnp.float32)
    w_ref[0] = jnp.concatenate(
        [r0 / norm, r1 / norm, r2 / norm, fzero], axis=0)


def _make_sc_gather(M, Dm, CH):
    info = plsc.get_sparse_core_info()
    NW = info.num_cores * info.num_subcores
    pts_per_w = M // NW
    n_chunks = pts_per_w // CH
    mesh = plsc.VectorSubcoreMesh(core_axis_name="c", subcore_axis_name="s")

    @functools.partial(
        pl.kernel, mesh=mesh,
        out_type=jax.ShapeDtypeStruct((M, Dm), jnp.float32),
        scratch_types=[
            pltpu.VMEM((3 * CH,), jnp.int32),
            pltpu.VMEM((3 * CH,), jnp.int32),
            pltpu.VMEM((3 * CH, Dm), jnp.float32),
            pltpu.VMEM((3 * CH, Dm), jnp.float32),
            pltpu.VMEM((3 * CH, 16), jnp.float32),
            pltpu.VMEM((3 * CH, 16), jnp.float32),
            pltpu.VMEM((CH, Dm), jnp.float32),
            pltpu.SemaphoreType.DMA,
            pltpu.SemaphoreType.DMA,
            pltpu.SemaphoreType.DMA,
            pltpu.SemaphoreType.DMA,
        ],
    )
    def sc_k(table_hbm, idx_hbm, w16_hbm, out_hbm,
             idx_v0, idx_v1, rows_v0, rows_v1, w_v0, w_v1, out_v,
             semr0, semr1, semw0, semw1):
        wid = jax.lax.axis_index("s") * info.num_cores + jax.lax.axis_index("c")
        pt_base0 = wid * pts_per_w
        idx_v = (idx_v0, idx_v1)
        rows_v = (rows_v0, rows_v1)
        w_v = (w_v0, w_v1)
        semr = (semr0, semr1)
        semw = (semw0, semw1)

        def fire(c, buf):
            ib = (pt_base0 + c * CH) * 3
            pltpu.sync_copy(idx_hbm.at[pl.ds(ib, 3 * CH)], idx_v[buf])
            pltpu.async_copy(table_hbm.at[idx_v[buf]], rows_v[buf], semr[buf])
            pltpu.async_copy(w16_hbm.at[pl.ds(ib, 3 * CH)], w_v[buf], semw[buf])

        fire(0, 0)

        def pair_body(gp, carry):
            for b in (0, 1):
                cur = 2 * gp + b
                pltpu.make_async_copy(
                    table_hbm.at[idx_v[b]], rows_v[b], semr[b]).wait()
                pltpu.make_async_copy(
                    w16_hbm.at[pl.ds(0, 3 * CH)], w_v[b], semw[b]).wait()

                nxt = cur + 1

                @pl.when(nxt < n_chunks)
                def _():
                    fire(nxt, 1 - b)

                rv = rows_v[b]
                wv = w_v[b]
                for p in range(CH):
                    w0 = wv[3 * p, :]
                    w1 = wv[3 * p + 1, :]
                    w2 = wv[3 * p + 2, :]
                    for c in range(Dm // 16):
                        sl = pl.ds(c * 16, 16)
                        acc = (rv[3 * p, sl] * w0
                               + rv[3 * p + 1, sl] * w1)
                        acc = acc + rv[3 * p + 2, sl] * w2
                        out_v[p, sl] = acc
                pltpu.sync_copy(
                    out_v, out_hbm.at[pl.ds(pt_base0 + cur * CH, CH)])
            return carry

        jax.lax.fori_loop(0, n_chunks // 2, pair_body, 0, unroll=False)

    return sc_k


def _fuse2_body(p1_ref, gi_ref, wfuse_ref, y_ref, stats_ref):
    b = pl.program_id(0)
    nb = pl.program_id(1)

    @pl.when(jnp.logical_and(b == 0, nb == 0))
    def _():
        stats_ref[...] = jnp.zeros_like(stats_ref)

    D = p1_ref.shape[1]
    wf = wfuse_ref[...]
    y = jax.lax.dot_general(
        wf[:, :D], p1_ref[0], (((1,), (0,)), ((), ())),
        preferred_element_type=jnp.float32)
    y = y + jax.lax.dot_general(
        wf[:, D:], gi_ref[...], (((1,), (1,)), ((), ())),
        preferred_element_type=jnp.float32)
    y_ref[0] = y
    stats_ref[...] += _stats_update(y)


def _mlp_body(cnt, xin_ref, stats_in_ref, g_ref, b_ref, w_ref,
              out_ref, stats_out_ref):
    b = pl.program_id(0)
    nb = pl.program_id(1)

    @pl.when(jnp.logical_and(b == 0, nb == 0))
    def _():
        stats_out_ref[...] = jnp.zeros_like(stats_out_ref)

    scale, off = _bn_coeffs(stats_in_ref, g_ref, b_ref, cnt)
    x = _gelu(xin_ref[0] * scale + off)
    t = jax.lax.dot_general(
        w_ref[...], x, (((1,), (0,)), ((), ())),
        preferred_element_type=jnp.float32)
    out_ref[0] = t
    stats_out_ref[...] += _stats_update(t)


def _final_body(cnt, y_ref, stats_y_ref, gf_ref, bf_ref,
                t2_ref, stats2_ref, g2_ref, b2_ref, out_ref):
    scale_f, off_f = _bn_coeffs(stats_y_ref, gf_ref, bf_ref, cnt)
    x = _gelu(y_ref[0] * scale_f + off_f)
    scale2, off2 = _bn_coeffs(stats2_ref, g2_ref, b2_ref, cnt)
    h = t2_ref[0] * scale2 + off2
    out_ref[0] = _gelu(h + x)


def kernel(xyz1, xyz2, points1, points2, W_fuse, g_fuse, b_fuse,
           W1, g1, b1, W2, g2, b2):
    B, N, _ = xyz1.shape
    S = xyz2.shape[1]
    D = points1.shape[1]
    C = W_fuse.shape[0]
    cnt = float(B * N)

    Nb = 1024 if N % 1024 == 0 else N
    NB = N // Nb

    xyz1t = jnp.transpose(xyz1, (0, 2, 1))
    xyz2t = jnp.transpose(xyz2, (0, 2, 1))
    col = lambda v: v.reshape(C, 1)

    f32 = jnp.float32
    grid = (B, NB)

    blk_n = pl.BlockSpec((1, C, Nb), lambda b, nb: (b, 0, nb))
    blk_full = lambda shape: pl.BlockSpec(shape, lambda b, nb: (0,) * len(shape))
    blk_stats = pl.BlockSpec((C, 8), lambda b, nb: (0, 0))

    idx_out, w_out = pl.pallas_call(
        _select_body,
        grid=grid,
        in_specs=[
            pl.BlockSpec((1, 3, Nb), lambda b, nb: (b, 0, nb)),
            pl.BlockSpec((1, 3, S), lambda b, nb: (b, 0, 0)),
        ],
        out_specs=[pl.BlockSpec((1, 8, Nb), lambda b, nb: (b, 0, nb)),
                   pl.BlockSpec((1, 8, Nb), lambda b, nb: (b, 0, nb))],
        out_shape=[
            jax.ShapeDtypeStruct((B, 8, N), jnp.int32),
            jax.ShapeDtypeStruct((B, 8, N), f32),
        ],
    )(xyz1t, xyz2t)

    M = B * N
    idx_flat = idx_out[:, :3, :].transpose(0, 2, 1).reshape(3 * M)
    w3 = w_out[:, :3, :].transpose(0, 2, 1).reshape(3 * M, 1)
    w16 = jnp.broadcast_to(w3, (3 * M, 16))
    table = jnp.transpose(points2, (0, 2, 1)).reshape(B * S, D)

    interp = _make_sc_gather(M, D, 16)(table, idx_flat, w16)

    y, stats_y = pl.pallas_call(
        _fuse2_body,
        grid=grid,
        in_specs=[
            blk_n,
            pl.BlockSpec((Nb, D), lambda b, nb: (b * NB + nb, 0)),
            blk_full((C, 2 * D)),
        ],
        out_specs=[blk_n, blk_stats],
        out_shape=[
            jax.ShapeDtypeStruct((B, C, N), f32),
            jax.ShapeDtypeStruct((C, 8), f32),
        ],
    )(points1, interp, W_fuse)

    Nb2 = 2048 if N % 2048 == 0 else Nb
    grid2 = (B, N // Nb2)
    blk_n2 = pl.BlockSpec((1, C, Nb2), lambda b, nb: (b, 0, nb))

    mlp = functools.partial(_mlp_body, cnt)
    mlp_call = lambda xin, stats, g, b, w: pl.pallas_call(
        mlp,
        grid=grid2,
        in_specs=[
            blk_n2,
            blk_stats,
            blk_full((C, 1)),
            blk_full((C, 1)),
            blk_full((C, C)),
        ],
        out_specs=[blk_n2, blk_stats],
        out_shape=[
            jax.ShapeDtypeStruct((B, C, N), f32),
            jax.ShapeDtypeStruct((C, 8), f32),
        ],
    )(xin, stats, col(g), col(b), w)

    t1, stats_1 = mlp_call(y, stats_y, g_fuse, b_fuse, W1)
    t2, stats_2 = mlp_call(t1, stats_1, g1, b1, W2)

    out = pl.pallas_call(
        functools.partial(_final_body, cnt),
        grid=grid2,
        in_specs=[
            blk_n2,
            blk_stats,
            blk_full((C, 1)),
            blk_full((C, 1)),
            blk_n2,
            blk_stats,
            blk_full((C, 1)),
            blk_full((C, 1)),
        ],
        out_specs=blk_n2,
        out_shape=jax.ShapeDtypeStruct((B, C, N), f32),
    )(y, stats_y, col(g_fuse), col(b_fuse), t2, stats_2, col(g2), col(b2))

    return out
